# baseline (speedup 1.0000x reference)
import functools

import jax
import jax.numpy as jnp
from jax import lax
from jax.experimental import pallas as pl
from jax.experimental.pallas import tpu as pltpu
from jax.experimental.pallas import tpu_sc as plsc

F = 16
L = 16
NC, NS = 2, 16
NW = NC * NS
B = 16384
BPW = B // NW
CHUNK = 128
NCHUNK = BPW // CHUNK
RPG = 128 // F


def _sc_embed_dot(user2d, item2d, ut128, it128):
    mesh = plsc.VectorSubcoreMesh(core_axis_name="c", subcore_axis_name="s")

    @functools.partial(
        pl.kernel,
        out_type=jax.ShapeDtypeStruct((B,), jnp.float32),
        mesh=mesh,
        scratch_types=[
            pltpu.VMEM((BPW,), jnp.int32),
            pltpu.VMEM((BPW,), jnp.int32),
            pltpu.VMEM((BPW,), jnp.int32),
            pltpu.VMEM((BPW,), jnp.int32),
            pltpu.VMEM((2, CHUNK, 128), jnp.float32),
            pltpu.VMEM((2, CHUNK, 128), jnp.float32),
            pltpu.VMEM((BPW,), jnp.float32),
            pltpu.SemaphoreType.DMA,
            pltpu.SemaphoreType.DMA,
        ],
        compiler_params=pltpu.CompilerParams(needs_layout_passes=False),
    )
    def k(user_hbm, item_hbm, ut_hbm, it_hbm, out_hbm,
          uidx, iidx, ugrow, igrow, urows, irows, outv, sem0, sem1):
        wid = lax.axis_index("s") * NC + lax.axis_index("c")
        pltpu.sync_copy(user_hbm.at[wid], uidx)
        pltpu.sync_copy(item_hbm.at[wid], iidx)
        for t in range(BPW // L):
            sl = pl.ds(t * L, L)
            ugrow[sl] = jnp.right_shift(uidx[sl], 3)
            igrow[sl] = jnp.right_shift(iidx[sl], 3)

        sems = (sem0, sem1)

        def fire(c):
            b = c & 1
            sl = pl.ds(c * CHUNK, CHUNK)
            return (
                pltpu.async_copy(ut_hbm.at[ugrow.at[sl]], urows.at[b], sems[b]),
                pltpu.async_copy(it_hbm.at[igrow.at[sl]], irows.at[b], sems[b]),
            )

        pending = fire(0)
        for c in range(NCHUNK):
            b = c & 1
            nxt = fire(c + 1) if c + 1 < NCHUNK else None
            for d in pending:
                d.wait()
            pending = nxt
            ub, ib = urows.at[b], irows.at[b]

            def blk_body(blk, carry, ub=ub, ib=ib, c=c):
                row = blk * L + lax.iota(jnp.int32, L)
                gsl = pl.ds(c * CHUNK + blk * L, L)
                ucol = jnp.bitwise_and(uidx[gsl], RPG - 1) * F
                icol = jnp.bitwise_and(iidx[gsl], RPG - 1) * F
                acc = jnp.zeros((L,), jnp.float32)
                for f in range(F):
                    acc = acc + (plsc.load_gather(ub, [row, ucol + f]) *
                                 plsc.load_gather(ib, [row, icol + f]))
                outv[gsl] = acc
                return carry

            lax.fori_loop(0, CHUNK // L, blk_body, 0)

        pltpu.sync_copy(outv, out_hbm.at[pl.ds(wid * BPW, BPW)])

    return k(user2d, item2d, ut128, it128)


def kernel(user, item, user_table, item_table):
    user2d = user.reshape(NW, BPW)
    item2d = item.reshape(NW, BPW)
    ut128 = user_table.reshape(-1, 128)
    it128 = item_table.reshape(-1, 128)
    out = _sc_embed_dot(user2d, item2d, ut128, it128)
    return out[:, None]


# speedup vs baseline: 5.9759x; 5.9759x over previous
import functools

import jax
import jax.numpy as jnp
from jax import lax
from jax.experimental import pallas as pl
from jax.experimental.pallas import tpu as pltpu
from jax.experimental.pallas import tpu_sc as plsc

F = 16
L = 16
NC, NS = 2, 16
NW = NC * NS
B = 16384
BPW = B // NW
CW = 8
NPAIR = BPW // L


def _sc_embed_dot(user2d, item2d, ut_t, it_t):
    mesh = plsc.VectorSubcoreMesh(core_axis_name="c", subcore_axis_name="s")

    @functools.partial(
        pl.kernel,
        out_type=jax.ShapeDtypeStruct((B,), jnp.float32),
        mesh=mesh,
        scratch_types=[
            pltpu.VMEM((BPW,), jnp.int32),
            pltpu.VMEM((BPW,), jnp.int32),
            pltpu.VMEM((2, CW, F, 128), jnp.float32),
            pltpu.VMEM((2, CW, F, 128), jnp.float32),
            pltpu.VMEM((L, L), jnp.float32),
            pltpu.VMEM((L, L), jnp.float32),
            pltpu.VMEM((BPW,), jnp.float32),
            pltpu.SemaphoreType.DMA,
            pltpu.SemaphoreType.DMA,
        ],
        compiler_params=pltpu.CompilerParams(needs_layout_passes=False),
    )
    def k(user_hbm, item_hbm, ut_hbm, it_hbm, out_hbm,
          uidx, iidx, ublk, iblk, ue, ie, outv, sem0, sem1):
        wid = lax.axis_index("s") * NC + lax.axis_index("c")
        pltpu.sync_copy(user_hbm.at[wid], uidx)
        pltpu.sync_copy(item_hbm.at[wid], iidx)
        sems = (sem0, sem1)
        rows = lax.iota(jnp.int32, L)

        def pair_vecs(pair):
            return uidx[pl.ds(pair * L, L)], iidx[pl.ds(pair * L, L)]

        def fire(p, uvec, ivec):
            for j in range(CW):
                cu = pl.multiple_of((uvec[p * CW + j] >> 7) * 128, 128)
                ci = pl.multiple_of((ivec[p * CW + j] >> 7) * 128, 128)
                pltpu.async_copy(
                    ut_hbm.at[:, pl.ds(cu, 128)], ublk.at[p, j], sems[p])
                pltpu.async_copy(
                    it_hbm.at[:, pl.ds(ci, 128)], iblk.at[p, j], sems[p])

        def drain(p):
            for j in range(CW):
                pltpu.make_async_copy(
                    ut_hbm.at[:, pl.ds(0, 128)], ublk.at[p, j], sems[p]).wait()
                pltpu.make_async_copy(
                    it_hbm.at[:, pl.ds(0, 128)], iblk.at[p, j], sems[p]).wait()

        def extract(p, uvec, ivec):
            for j in range(CW):
                ucol = jnp.full((L,), uvec[p * CW + j] & 127, dtype=jnp.int32)
                icol = jnp.full((L,), ivec[p * CW + j] & 127, dtype=jnp.int32)
                r = p * CW + j
                ue[r, :] = plsc.load_gather(ublk.at[p, j], [rows, ucol])
                ie[r, :] = plsc.load_gather(iblk.at[p, j], [rows, icol])

        def compute_store(pair):
            acc = jnp.zeros((L,), jnp.float32)
            for f in range(F):
                col = jnp.full((L,), f, dtype=jnp.int32)
                acc = acc + (plsc.load_gather(ue, [rows, col]) *
                             plsc.load_gather(ie, [rows, col]))
            outv[pl.ds(pair * L, L)] = acc

        uv0, iv0 = pair_vecs(0)
        fire(0, uv0, iv0)
        fire(1, uv0, iv0)

        def body(pair, carry):
            uvec, ivec = pair_vecs(pair)
            uvn, ivn = pair_vecs(pair + 1)
            drain(0)
            extract(0, uvec, ivec)
            fire(0, uvn, ivn)
            drain(1)
            extract(1, uvec, ivec)
            fire(1, uvn, ivn)
            compute_store(pair)
            return carry

        lax.fori_loop(0, NPAIR - 1, body, 0)

        uvl, ivl = pair_vecs(NPAIR - 1)
        drain(0)
        extract(0, uvl, ivl)
        drain(1)
        extract(1, uvl, ivl)
        compute_store(NPAIR - 1)

        pltpu.sync_copy(outv, out_hbm.at[pl.ds(wid * BPW, BPW)])

    return k(user2d, item2d, ut_t, it_t)


def kernel(user, item, user_table, item_table):
    user2d = user.reshape(NW, BPW)
    item2d = item.reshape(NW, BPW)
    out = _sc_embed_dot(user2d, item2d, user_table.T, item_table.T)
    return out[:, None]
